# baseline (speedup 1.0000x reference)
import functools

import jax
import jax.numpy as jnp
from jax import lax
from jax.experimental import pallas as pl
from jax.experimental.pallas import tpu as pltpu
from jax.experimental.pallas import tpu_sc as plsc

N_ROWS = 100000
D = 128
B = 1024
NC = 2
NS = 16
L = 16
DH = D // NC
BLK = 128
NBLK = N_ROWS // BLK
TAIL = N_ROWS - NBLK * BLK
BLK_ITERS = (NBLK + NS - 1) // NS
NPAD = BLK_ITERS * NS * BLK
SEG_PER_TILE = B // NS
X2_PER_W = B // (NC * NS)

_mesh = plsc.VectorSubcoreMesh(core_axis_name="c", subcore_axis_name="s")


@functools.partial(
    pl.kernel,
    out_type=jax.ShapeDtypeStruct((B, 2 * D), jnp.float32),
    mesh=_mesh,
    compiler_params=pltpu.CompilerParams(use_tc_tiling_on_sc=False,
                                         needs_layout_passes=False),
    scratch_types=[
        pltpu.VMEM((BLK_ITERS, BLK), jnp.int32),
        pltpu.VMEM((BLK, DH), jnp.float32),
        pltpu.VMEM((BLK, DH), jnp.float32),
        pltpu.VMEM((B,), jnp.float32),
        pltpu.VMEM((SEG_PER_TILE, DH), jnp.float32),
        pltpu.VMEM((SEG_PER_TILE, L), jnp.float32),
        pltpu.VMEM((NS, SEG_PER_TILE), jnp.float32),
        pltpu.VMEM((X2_PER_W, D), jnp.float32),
        pltpu.VMEM((TAIL,), jnp.int32),
        pltpu.VMEM((TAIL, DH), jnp.float32),
        pltpu.VMEM_SHARED((B, DH), jnp.float32),
        pltpu.VMEM_SHARED((NS, B), jnp.float32),
        pltpu.SemaphoreType.DMA,
        pltpu.SemaphoreType.DMA,
        pltpu.SemaphoreType.DMA,
        pltpu.SemaphoreType.DMA,
        pltpu.SemaphoreType.DMA,
        pltpu.SemaphoreType.DMA,
    ],
)
def _seg_mean_concat(x1_hbm, b1_hbm, b1_3d_hbm, x2_hbm, out_hbm,
                     ids2d, rows_a, rows_b, lcnt, accv, cntv, cstage,
                     x2v, idx_t, rows_t,
                     acc_sh, cnt_sh, sid, sra, srb, ssa, ssb, sx2):
    c = lax.axis_index("c")
    s = lax.axis_index("s")
    col0 = c * DH
    seg0 = s * SEG_PER_TILE

    zeros16 = jnp.zeros((L,), jnp.float32)

    row_slots = (rows_a, rows_b)
    rsem_slots = (sra, srb)
    ssem_slots = (ssa, ssb)

    def _start_rows(bid, slot):
        pltpu.async_copy(
            x1_hbm.at[pl.ds(bid * BLK, BLK), pl.ds(col0, DH)],
            row_slots[slot], rsem_slots[slot])

    def _wait_rows(slot):
        pltpu.make_async_copy(
            x1_hbm.at[pl.ds(0, BLK), pl.ds(col0, DH)],
            row_slots[slot], rsem_slots[slot]).wait()

    def _start_scatter(k, slot):
        pltpu.async_copy(row_slots[slot], acc_sh.at[ids2d.at[k]],
                         ssem_slots[slot], add=True)

    def _wait_scatter(k, slot):
        pltpu.make_async_copy(row_slots[slot], acc_sh.at[ids2d.at[k]],
                              ssem_slots[slot]).wait()

    def _hist_chunk(ids):
        cnt_i, last = plsc.scan_count(ids)
        plsc.addupdate_scatter(lcnt, [ids], cnt_i.astype(jnp.float32),
                               mask=last)

    w = c * NS + s
    r0 = w * X2_PER_W
    pltpu.async_copy(x2_hbm.at[pl.ds(r0, X2_PER_W)], x2v, sx2)
    pltpu.async_copy(b1_3d_hbm.at[:, s, :], ids2d, sid)
    _start_rows(s, 0)

    def _zero_cnt(r, _):
        lcnt[pl.ds(r * L, L)] = zeros16
        return 0

    lax.fori_loop(0, B // L, _zero_cnt, 0)

    def _zero_row(r, _):
        for cc in range(DH // L):
            accv[r, pl.ds(cc * L, L)] = zeros16
        return 0

    lax.fori_loop(0, SEG_PER_TILE, _zero_row, 0)
    pltpu.sync_copy(accv, acc_sh.at[pl.ds(seg0, SEG_PER_TILE)])
    pltpu.make_async_copy(b1_3d_hbm.at[:, s, :], ids2d, sid).wait()
    plsc.subcore_barrier()

    def _pair(k2, _):
        for b in range(2):
            k = 2 * k2 + b
            bid = s + NS * k
            bid_next = bid + NS

            @pl.when(jnp.logical_and(bid_next < NBLK, bid >= s + NS))
            def _():
                _wait_scatter(k - 1, 1 - b)

            @pl.when(bid_next < NBLK)
            def _():
                _start_rows(bid_next, 1 - b)

            @pl.when(bid < NBLK)
            def _():
                _wait_rows(b)
                _start_scatter(k, b)
                for g in range(BLK // L):
                    _hist_chunk(ids2d[k, pl.ds(g * L, L)])

        return 0

    lax.fori_loop(0, (BLK_ITERS + 1) // 2, _pair, 0)
    _wait_scatter(0, 0)
    _wait_scatter(0, 1)

    @pl.when(s == NS - 1)
    def _tail():
        tstart = NBLK * BLK
        pltpu.sync_copy(b1_hbm.at[pl.ds(tstart, TAIL)], idx_t)
        pltpu.sync_copy(x1_hbm.at[pl.ds(tstart, TAIL), pl.ds(col0, DH)],
                        rows_t)
        pltpu.sync_copy(rows_t, acc_sh.at[idx_t], add=True)
        for g in range(TAIL // L):
            _hist_chunk(idx_t[pl.ds(g * L, L)])

    pltpu.sync_copy(lcnt, cnt_sh.at[s])
    pltpu.make_async_copy(x2_hbm.at[pl.ds(r0, X2_PER_W)], x2v, sx2).wait()
    pltpu.sync_copy(x2v, out_hbm.at[pl.ds(r0, X2_PER_W), pl.ds(D, D)])
    plsc.subcore_barrier()

    pltpu.sync_copy(acc_sh.at[pl.ds(seg0, SEG_PER_TILE)], accv)
    pltpu.sync_copy(cnt_sh.at[:, pl.ds(seg0, SEG_PER_TILE)], cstage)

    lanes = lax.iota(jnp.int32, L)
    for g in range(SEG_PER_TILE // L):
        tot = cstage[0, pl.ds(g * L, L)]
        for r in range(1, NS):
            tot = tot + cstage[r, pl.ds(g * L, L)]
        rec = 1.0 / jnp.maximum(tot, 1.0)
        rows_idx = g * L + lanes
        for j in range(L):
            cols_idx = jnp.full((L,), j, jnp.int32)
            plsc.store_scatter(cntv, [rows_idx, cols_idx], rec)

    def _finalize_row(r, _):
        rec = cntv[r, :]
        for cc in range(DH // L):
            accv[r, pl.ds(cc * L, L)] = accv[r, pl.ds(cc * L, L)] * rec
        return 0

    lax.fori_loop(0, SEG_PER_TILE, _finalize_row, 0)
    pltpu.sync_copy(accv, out_hbm.at[pl.ds(seg0, SEG_PER_TILE),
                                     pl.ds(col0, DH)])


def kernel(x1, batch1, x2):
    b1 = batch1.astype(jnp.int32)
    b1_3d = jnp.pad(b1, (0, NPAD - N_ROWS)).reshape(BLK_ITERS, NS, BLK)
    return _seg_mean_concat(x1.astype(jnp.float32), b1, b1_3d,
                            x2.astype(jnp.float32))


# speedup vs baseline: 1.0978x; 1.0978x over previous
import functools

import jax
import jax.numpy as jnp
from jax import lax
from jax.experimental import pallas as pl
from jax.experimental.pallas import tpu as pltpu
from jax.experimental.pallas import tpu_sc as plsc

N_ROWS = 100000
D = 128
B = 1024
NC = 2
NS = 16
L = 16
DH = D // NC
BLK = 128
NBLK = N_ROWS // BLK
TAIL = N_ROWS - NBLK * BLK
BLK_ITERS = (NBLK + NS - 1) // NS
SEG_PER_TILE = B // NS
X2_PER_W = B // (NC * NS)

_mesh = plsc.VectorSubcoreMesh(core_axis_name="c", subcore_axis_name="s")


@functools.partial(
    pl.kernel,
    out_type=jax.ShapeDtypeStruct((B, 2 * D), jnp.float32),
    mesh=_mesh,
    compiler_params=pltpu.CompilerParams(use_tc_tiling_on_sc=False,
                                         needs_layout_passes=False),
    scratch_types=[
        pltpu.VMEM((BLK,), jnp.int32),
        pltpu.VMEM((BLK,), jnp.int32),
        pltpu.VMEM((BLK,), jnp.int32),
        pltpu.VMEM((BLK, DH), jnp.float32),
        pltpu.VMEM((BLK, DH), jnp.float32),
        pltpu.VMEM((BLK, DH), jnp.float32),
        pltpu.VMEM((B,), jnp.float32),
        pltpu.VMEM((SEG_PER_TILE, DH), jnp.float32),
        pltpu.VMEM((SEG_PER_TILE, L), jnp.float32),
        pltpu.VMEM((NS, SEG_PER_TILE), jnp.float32),
        pltpu.VMEM((X2_PER_W, D), jnp.float32),
        pltpu.VMEM((TAIL,), jnp.int32),
        pltpu.VMEM((TAIL, DH), jnp.float32),
        pltpu.VMEM_SHARED((B, DH), jnp.float32),
        pltpu.VMEM_SHARED((NS, B), jnp.float32),
        pltpu.SemaphoreType.DMA,
        pltpu.SemaphoreType.DMA,
        pltpu.SemaphoreType.DMA,
        pltpu.SemaphoreType.DMA,
        pltpu.SemaphoreType.DMA,
        pltpu.SemaphoreType.DMA,
        pltpu.SemaphoreType.DMA,
        pltpu.SemaphoreType.DMA,
        pltpu.SemaphoreType.DMA,
        pltpu.SemaphoreType.DMA,
    ],
)
def _seg_mean_concat(x1_hbm, b1_hbm, x2_hbm, out_hbm,
                     idx_a, idx_b, idx_c, rows_a, rows_b, rows_c,
                     lcnt, accv, cntv, cstage, x2v, idx_t, rows_t,
                     acc_sh, cnt_sh, sia, sib, sic, sra, srb, src,
                     ssa, ssb, ssc, sx2):
    c = lax.axis_index("c")
    s = lax.axis_index("s")
    col0 = c * DH
    seg0 = s * SEG_PER_TILE

    zeros16 = jnp.zeros((L,), jnp.float32)

    idx_slots = (idx_a, idx_b, idx_c)
    row_slots = (rows_a, rows_b, rows_c)
    isem_slots = (sia, sib, sic)
    rsem_slots = (sra, srb, src)
    ssem_slots = (ssa, ssb, ssc)

    def _start_loads(bid, slot):
        start = bid * BLK
        pltpu.async_copy(b1_hbm.at[pl.ds(start, BLK)], idx_slots[slot],
                         isem_slots[slot])
        pltpu.async_copy(x1_hbm.at[pl.ds(start, BLK), pl.ds(col0, DH)],
                         row_slots[slot], rsem_slots[slot])

    def _wait_loads(bid, slot):
        start = bid * BLK
        pltpu.make_async_copy(b1_hbm.at[pl.ds(start, BLK)], idx_slots[slot],
                              isem_slots[slot]).wait()
        pltpu.make_async_copy(x1_hbm.at[pl.ds(start, BLK), pl.ds(col0, DH)],
                              row_slots[slot], rsem_slots[slot]).wait()

    def _start_scatter(slot):
        pltpu.async_copy(row_slots[slot], acc_sh.at[idx_slots[slot]],
                         ssem_slots[slot], add=True)

    def _wait_scatter(slot):
        pltpu.make_async_copy(row_slots[slot], acc_sh.at[idx_slots[slot]],
                              ssem_slots[slot]).wait()

    def _hist(idx_ref, nchunks):
        for g in range(nchunks):
            ids = idx_ref[pl.ds(g * L, L)]
            cnt_i, last = plsc.scan_count(ids)
            plsc.addupdate_scatter(lcnt, [ids], cnt_i.astype(jnp.float32),
                                   mask=last)

    w = c * NS + s
    r0 = w * X2_PER_W
    pltpu.async_copy(x2_hbm.at[pl.ds(r0, X2_PER_W)], x2v, sx2)
    _start_loads(s, 0)

    def _zero_cnt(r, _):
        lcnt[pl.ds(r * L, L)] = zeros16
        return 0

    lax.fori_loop(0, B // L, _zero_cnt, 0)

    def _zero_row(r, _):
        for cc in range(DH // L):
            accv[r, pl.ds(cc * L, L)] = zeros16
        return 0

    lax.fori_loop(0, SEG_PER_TILE, _zero_row, 0)
    pltpu.sync_copy(accv, acc_sh.at[pl.ds(seg0, SEG_PER_TILE)])
    plsc.subcore_barrier()

    def _triple(k3, _):
        for b in range(3):
            j3 = 3 * k3 + b
            bid = s + NS * j3
            bid_next = bid + NS
            nxt = (b + 1) % 3

            @pl.when(jnp.logical_and(bid_next < NBLK, bid >= s + 2 * NS))
            def _():
                _wait_scatter(nxt)

            @pl.when(bid_next < NBLK)
            def _():
                _start_loads(bid_next, nxt)

            @pl.when(bid < NBLK)
            def _():
                _wait_loads(bid, b)
                _start_scatter(b)
                _hist(idx_slots[b], BLK // L)

        return 0

    lax.fori_loop(0, (BLK_ITERS + 2) // 3, _triple, 0)
    _wait_scatter(0)
    _wait_scatter(1)
    _wait_scatter(2)

    @pl.when(s == NS - 1)
    def _tail():
        tstart = NBLK * BLK
        pltpu.sync_copy(b1_hbm.at[pl.ds(tstart, TAIL)], idx_t)
        pltpu.sync_copy(x1_hbm.at[pl.ds(tstart, TAIL), pl.ds(col0, DH)],
                        rows_t)
        pltpu.sync_copy(rows_t, acc_sh.at[idx_t], add=True)
        _hist(idx_t, TAIL // L)

    pltpu.sync_copy(lcnt, cnt_sh.at[s])
    pltpu.make_async_copy(x2_hbm.at[pl.ds(r0, X2_PER_W)], x2v, sx2).wait()
    pltpu.sync_copy(x2v, out_hbm.at[pl.ds(r0, X2_PER_W), pl.ds(D, D)])
    plsc.subcore_barrier()

    pltpu.sync_copy(acc_sh.at[pl.ds(seg0, SEG_PER_TILE)], accv)
    pltpu.sync_copy(cnt_sh.at[:, pl.ds(seg0, SEG_PER_TILE)], cstage)

    lanes = lax.iota(jnp.int32, L)
    for g in range(SEG_PER_TILE // L):
        tot = cstage[0, pl.ds(g * L, L)]
        for r in range(1, NS):
            tot = tot + cstage[r, pl.ds(g * L, L)]
        rec = 1.0 / jnp.maximum(tot, 1.0)
        rows_idx = g * L + lanes
        for j in range(L):
            cols_idx = jnp.full((L,), j, jnp.int32)
            plsc.store_scatter(cntv, [rows_idx, cols_idx], rec)

    def _finalize_row(r, _):
        rec = cntv[r, :]
        for cc in range(DH // L):
            accv[r, pl.ds(cc * L, L)] = accv[r, pl.ds(cc * L, L)] * rec
        return 0

    lax.fori_loop(0, SEG_PER_TILE, _finalize_row, 0)
    pltpu.sync_copy(accv, out_hbm.at[pl.ds(seg0, SEG_PER_TILE),
                                     pl.ds(col0, DH)])


def kernel(x1, batch1, x2):
    return _seg_mean_concat(x1.astype(jnp.float32),
                            batch1.astype(jnp.int32),
                            x2.astype(jnp.float32))
